# baseline (speedup 1.0000x reference)
import jax
import jax.numpy as jnp
from jax import lax
from jax.experimental import pallas as pl
from jax.experimental.pallas import tpu as pltpu
from jax.experimental.pallas import tpu_sc as plsc

_NC = 2
_NS = 16
_L = 16

_C = 96
_CHUNK = 1792
_NBUF = 2


def _sc_gather(idx_hbm, x_hbm, out_hbm, idxf_v, sel_v, idx_bufs, row_bufs,
               gsems, ssems, rows_per_w, n_groups):
    cid = lax.axis_index("c")
    sid = lax.axis_index("s")
    wid = sid * _NC + cid

    pltpu.sync_copy(idx_hbm, idxf_v)
    zeros = jnp.zeros((_L,), jnp.int32)
    for k in range(_C // _L):
        sel_v[pl.ds(_L * k, _L)] = zeros
    iota = lax.iota(jnp.int32, _L)
    ones = jnp.ones((_L,), jnp.int32)
    offset = zeros
    for k in range(_C // _L):
        v = idxf_v[pl.ds(_L * k, _L)]
        m = v != jnp.zeros((_L,), jnp.float32)
        mi = jnp.where(m, ones, zeros)
        pos = plsc.cumsum(mi) - ones + offset
        plsc.store_scatter(sel_v, [pos], iota + jnp.full((_L,), _L * k, jnp.int32), mask=m)
        offset = offset + lax.broadcast(jnp.sum(mi), (_L,))

    kpc = 50176 // _CHUNK
    base_row = wid * rows_per_w

    def outer(go, carry):
        for b in range(_NBUF):
            g = go * _NBUF + b
            ob = base_row + g * _L
            o = lax.broadcast(ob, (_L,)) + iota
            j = lax.rem(o // jnp.full((_L,), kpc, jnp.int32),
                        jnp.full((_L,), _C, jnp.int32))
            sel_j = plsc.load_gather(sel_v, [j])
            src = o + (sel_j - j) * jnp.full((_L,), kpc, jnp.int32)

            @pl.when(go > 0)
            def _():
                pltpu.make_async_copy(
                    row_bufs[b], out_hbm.at[pl.ds(0, _L)], ssems[b]
                ).wait()

            idx_bufs[b][...] = src
            pltpu.async_copy(x_hbm.at[idx_bufs[b]], row_bufs[b], gsems[b]).wait()
            pltpu.async_copy(row_bufs[b], out_hbm.at[pl.ds(ob, _L)], ssems[b])
        return carry

    lax.fori_loop(0, n_groups // _NBUF, outer, jnp.int32(0))
    for b in range(_NBUF):
        pltpu.make_async_copy(
            row_bufs[b], out_hbm.at[pl.ds(0, _L)], ssems[b]
        ).wait()


@jax.jit
def kernel(input_tensor, indexes):
    b, c, h, w = input_tensor.shape
    hw = h * w
    n_rows = b * c * hw // _CHUNK
    n_workers = _NC * _NS
    rows_per_w = n_rows // n_workers
    n_groups = rows_per_w // _L

    x = input_tensor.reshape(n_rows, _CHUNK)
    mesh = plsc.VectorSubcoreMesh(
        core_axis_name="c", subcore_axis_name="s",
        num_cores=_NC, num_subcores=_NS,
    )

    def body(idx_hbm, x_hbm, out_hbm, idxf_v, sel_v, i0, i1, r0, r1,
             g0, g1, s0, s1):
        _sc_gather(idx_hbm, x_hbm, out_hbm, idxf_v, sel_v, [i0, i1],
                   [r0, r1], [g0, g1], [s0, s1], rows_per_w, n_groups)

    out = pl.kernel(
        body,
        out_type=jax.ShapeDtypeStruct((n_rows, _CHUNK), jnp.float32),
        mesh=mesh,
        compiler_params=pltpu.CompilerParams(needs_layout_passes=False),
        scratch_types=[
            pltpu.VMEM((c,), jnp.float32),
            pltpu.VMEM((c,), jnp.int32),
            pltpu.VMEM((_L,), jnp.int32),
            pltpu.VMEM((_L,), jnp.int32),
            pltpu.VMEM((_L, _CHUNK), jnp.float32),
            pltpu.VMEM((_L, _CHUNK), jnp.float32),
            pltpu.SemaphoreType.DMA,
            pltpu.SemaphoreType.DMA,
            pltpu.SemaphoreType.DMA,
            pltpu.SemaphoreType.DMA,
        ],
    )(indexes, x)
    return out.reshape(b, c, h, w)


# speedup vs baseline: 1.0872x; 1.0872x over previous
import jax
import jax.numpy as jnp
from jax import lax
from jax.experimental import pallas as pl
from jax.experimental.pallas import tpu as pltpu
from jax.experimental.pallas import tpu_sc as plsc

_NC = 2
_NS = 16
_L = 16

_C = 96
_NBUF = 4
_HALF = 25088


def _sc_gather(idx_hbm, x_hbm, out_hbm, idxf_v, sel_v, bufs, gsems, ssems,
               slabs_per_w, hw):
    cid = lax.axis_index("c")
    sid = lax.axis_index("s")
    wid = sid * _NC + cid

    pltpu.sync_copy(idx_hbm, idxf_v)
    zeros = jnp.zeros((_L,), jnp.int32)
    for k in range(_C // _L):
        sel_v[pl.ds(_L * k, _L)] = zeros
    iota = lax.iota(jnp.int32, _L)
    ones = jnp.ones((_L,), jnp.int32)
    offset = zeros
    for k in range(_C // _L):
        v = idxf_v[pl.ds(_L * k, _L)]
        m = v != jnp.zeros((_L,), jnp.float32)
        mi = jnp.where(m, ones, zeros)
        pos = plsc.cumsum(mi) - ones + offset
        plsc.store_scatter(sel_v, [pos], iota + jnp.full((_L,), _L * k, jnp.int32), mask=m)
        offset = offset + lax.broadcast(jnp.sum(mi), (_L,))

    halves_per_slab = hw // _HALF
    n_t = slabs_per_w * halves_per_slab
    base_slab = wid * slabs_per_w

    def src_dst(t):
        slab_local = t // halves_per_slab
        half = t - slab_local * halves_per_slab
        s_global = base_slab + slab_local
        bi = s_global // _C
        j = s_global - bi * _C
        sel_vec = plsc.load_gather(sel_v, [lax.broadcast(j, (_L,))])
        sj = jnp.max(sel_vec)
        src_slab = bi * _C + sj
        off = half * _HALF
        return (x_hbm.at[pl.ds(src_slab, 1), pl.ds(off, _HALF)],
                out_hbm.at[pl.ds(s_global, 1), pl.ds(off, _HALF)])

    def outer(go, carry):
        for b in range(_NBUF):
            t = go * _NBUF + b
            src, _ = src_dst(t)

            @pl.when(go > 0)
            def _():
                pltpu.make_async_copy(
                    bufs[b], out_hbm.at[pl.ds(0, 1), pl.ds(0, _HALF)], ssems[b]
                ).wait()

            pltpu.async_copy(src, bufs[b], gsems[b])
        for b in range(_NBUF):
            t = go * _NBUF + b
            _, dst = src_dst(t)
            pltpu.make_async_copy(
                x_hbm.at[pl.ds(0, 1), pl.ds(0, _HALF)], bufs[b], gsems[b]
            ).wait()
            pltpu.async_copy(bufs[b], dst, ssems[b])
        return carry

    lax.fori_loop(0, n_t // _NBUF, outer, jnp.int32(0))
    for b in range(_NBUF):
        pltpu.make_async_copy(
            bufs[b], out_hbm.at[pl.ds(0, 1), pl.ds(0, _HALF)], ssems[b]
        ).wait()


@jax.jit
def kernel(input_tensor, indexes):
    b, c, h, w = input_tensor.shape
    hw = h * w
    n_slabs = b * c
    n_workers = _NC * _NS
    slabs_per_w = n_slabs // n_workers

    x = input_tensor.reshape(n_slabs, hw)
    mesh = plsc.VectorSubcoreMesh(
        core_axis_name="c", subcore_axis_name="s",
        num_cores=_NC, num_subcores=_NS,
    )

    def body(idx_hbm, x_hbm, out_hbm, idxf_v, sel_v, b0, b1, b2, b3,
             g0, g1, g2, g3, s0, s1, s2, s3):
        _sc_gather(idx_hbm, x_hbm, out_hbm, idxf_v, sel_v,
                   [b0, b1, b2, b3], [g0, g1, g2, g3], [s0, s1, s2, s3],
                   slabs_per_w, hw)

    out = pl.kernel(
        body,
        out_type=jax.ShapeDtypeStruct((n_slabs, hw), jnp.float32),
        mesh=mesh,
        compiler_params=pltpu.CompilerParams(needs_layout_passes=False),
        scratch_types=(
            [pltpu.VMEM((c,), jnp.float32), pltpu.VMEM((c,), jnp.int32)]
            + [pltpu.VMEM((1, _HALF), jnp.float32)] * _NBUF
            + [pltpu.SemaphoreType.DMA] * (2 * _NBUF)
        ),
    )(indexes, x)
    return out.reshape(b, c, h, w)
